# baseline (speedup 1.0000x reference)
import functools

import jax
import jax.numpy as jnp
from jax.experimental import pallas as pl
from jax.experimental.pallas import tpu as pltpu

_MASK_IDX = 5
_BM = 512


def _patch_mm(x_ref, w_ref, b_ref, acts_ref, o_ref, *, blocks_per_batch):
    y = jnp.dot(x_ref[...], w_ref[...], preferred_element_type=jnp.float32)
    o_ref[...] = y + b_ref[...]

    @pl.when(pl.program_id(0) % blocks_per_batch == 0)
    def _():
        o_ref[_MASK_IDX, :] = acts_ref[0]


def kernel(x, W, b, acts):
    B, S, D = x.shape
    xf = x.reshape(B * S, D)
    b2 = b.reshape(1, D)
    acts2 = acts.reshape(1, D)
    bm = _BM
    grid = (B * S // bm,)
    out = pl.pallas_call(
        functools.partial(_patch_mm, blocks_per_batch=S // bm),
        grid=grid,
        in_specs=[
            pl.BlockSpec((bm, D), lambda i: (i, 0)),
            pl.BlockSpec((D, D), lambda i: (0, 0)),
            pl.BlockSpec((1, D), lambda i: (0, 0)),
            pl.BlockSpec((1, D), lambda i: (0, 0)),
        ],
        out_specs=pl.BlockSpec((bm, D), lambda i: (i, 0)),
        out_shape=jax.ShapeDtypeStruct((B * S, D), jnp.float32),
        compiler_params=pltpu.CompilerParams(
            dimension_semantics=("arbitrary",),
        ),
    )(xf, W, b2, acts2)
    return out.reshape(B, S, D)


# speedup vs baseline: 1.0003x; 1.0003x over previous
import functools

import jax
import jax.numpy as jnp
from jax.experimental import pallas as pl
from jax.experimental.pallas import tpu as pltpu

_MASK_IDX = 5
_BM = 512


def _patch_mm(x_ref, w_ref, b_ref, acts_ref, o_ref, *, blocks_per_batch):
    y = jnp.dot(
        x_ref[...].astype(jnp.bfloat16),
        w_ref[...].astype(jnp.bfloat16),
        preferred_element_type=jnp.float32,
    )
    o_ref[...] = y + b_ref[...]

    @pl.when(pl.program_id(0) % blocks_per_batch == 0)
    def _():
        o_ref[_MASK_IDX, :] = acts_ref[0]


def kernel(x, W, b, acts):
    B, S, D = x.shape
    xf = x.reshape(B * S, D)
    b2 = b.reshape(1, D)
    acts2 = acts.reshape(1, D)
    bm = _BM
    grid = (B * S // bm,)
    out = pl.pallas_call(
        functools.partial(_patch_mm, blocks_per_batch=S // bm),
        grid=grid,
        in_specs=[
            pl.BlockSpec((bm, D), lambda i: (i, 0)),
            pl.BlockSpec((D, D), lambda i: (0, 0)),
            pl.BlockSpec((1, D), lambda i: (0, 0)),
            pl.BlockSpec((1, D), lambda i: (0, 0)),
        ],
        out_specs=pl.BlockSpec((bm, D), lambda i: (i, 0)),
        out_shape=jax.ShapeDtypeStruct((B * S, D), jnp.float32),
        compiler_params=pltpu.CompilerParams(
            dimension_semantics=("arbitrary",),
        ),
    )(xf, W, b2, acts2)
    return out.reshape(B, S, D)
